# Initial kernel scaffold; baseline (speedup 1.0000x reference)
import jax
import jax.numpy as jnp
from jax.experimental import pallas as pl


def kernel(x, tables):
    raise NotImplementedError("write your pallas kernel here")


# speedup vs baseline: 1.2129x; 1.2129x over previous
import functools

import jax
import jax.numpy as jnp
from jax import lax
from jax.experimental import pallas as pl
from jax.experimental.pallas import tpu as pltpu
from jax.experimental.pallas import tpu_sc as plsc

NUM_FIELDS = 26
VOCAB = 100000
EMB_DIM = 32
BATCH = 16384

NC, NS, L = 2, 16, 16
NW = NC * NS
N_ROWS = BATCH * NUM_FIELDS
PER_W = N_ROWS // NW
NCH = 16
CR = PER_W // NCH
NBUF = 4
PAT = 208

_mesh = plsc.VectorSubcoreMesh(
    core_axis_name="c", subcore_axis_name="s",
    num_cores=NC, num_subcores=NS)


@functools.partial(
    pl.kernel,
    out_type=jax.ShapeDtypeStruct((N_ROWS, EMB_DIM), jnp.float32),
    mesh=_mesh,
    compiler_params=pltpu.CompilerParams(use_tc_tiling_on_sc=False),
    scratch_types=[
        pltpu.VMEM((PER_W,), jnp.int32),
        pltpu.VMEM((PAT,), jnp.int32),
        *([pltpu.VMEM((CR, EMB_DIM), jnp.float32)] * NBUF),
        *([pltpu.SemaphoreType.DMA] * (2 * NBUF)),
    ],
)
def _embed_gather(x_hbm, tbl_hbm, out_hbm, idx_v, patt_v,
                  b0, b1, b2, b3, g0, g1, g2, g3, w0, w1, w2, w3):
    bufs = (b0, b1, b2, b3)
    gsems = (g0, g1, g2, g3)
    wsems = (w0, w1, w2, w3)
    wid = lax.axis_index("s") * NC + lax.axis_index("c")
    base = wid * PER_W

    pltpu.sync_copy(x_hbm.at[pl.ds(base, PER_W)], idx_v)

    for j in range(PAT // L):
        lane = lax.iota(jnp.int32, L) + (j * L)
        patt_v[pl.ds(j * L, L)] = lax.rem(lane, NUM_FIELDS) * VOCAB

    def add_offsets(g, carry):
        off = g * PAT
        for j in range(PAT // L):
            sl = pl.ds(off + j * L, L)
            idx_v[sl] = idx_v[sl] + patt_v[pl.ds(j * L, L)]
        return carry
    lax.fori_loop(0, PER_W // PAT, add_offsets, 0)

    def gdesc(c, b):
        return pltpu.make_async_copy(
            tbl_hbm.at[idx_v.at[pl.ds(c * CR, CR)]], bufs[b], gsems[b])

    def wdesc(c, b):
        return pltpu.make_async_copy(
            bufs[b], out_hbm.at[pl.ds(base + c * CR, CR)], wsems[b])

    for b in range(NBUF - 1):
        gdesc(b, b).start()

    def group(gi, carry):
        for b in range(NBUF):
            c = gi * NBUF + b
            gdesc(c, b).wait()
            wdesc(c, b).start()
            bb = (b + NBUF - 1) % NBUF

            @pl.when(c >= 1)
            def _():
                wdesc(c - 1, bb).wait()

            @pl.when(c + NBUF - 1 < NCH)
            def _():
                gdesc(c + NBUF - 1, bb).start()
        return carry
    lax.fori_loop(0, NCH // NBUF, group, 0)

    wdesc(NCH - 1, (NCH - 1) % NBUF).wait()


def kernel(x, tables):
    tbl = tables.reshape(NUM_FIELDS * VOCAB, EMB_DIM)
    xf = x.reshape(N_ROWS)
    out = _embed_gather(xf, tbl)
    return out.reshape(BATCH, NUM_FIELDS * EMB_DIM)
